# Initial kernel scaffold; baseline (speedup 1.0000x reference)
import jax
import jax.numpy as jnp
from jax.experimental import pallas as pl


def kernel(x, codes, codebook):
    raise NotImplementedError("write your pallas kernel here")


# speedup vs baseline: 7.0351x; 7.0351x over previous
import functools

import jax
import jax.numpy as jnp
from jax import lax
from jax.experimental import pallas as pl
from jax.experimental.pallas import tpu as pltpu
from jax.experimental.pallas import tpu_sc as plsc

_NC, _NS = 2, 16
_NW = _NC * _NS
_D = 64
_M = 8
_CHUNK = 128
_FMT = None

_FMT = plsc.PackFormat.INTERLEAVED


def _pack_codebook(codebook):
    cb16 = codebook.astype(jnp.bfloat16)
    g0 = jnp.stack([cb16[:, 0:16], cb16[:, 16:32]], axis=-1)
    g1 = jnp.stack([cb16[:, 32:48], cb16[:, 48:64]], axis=-1)
    return jnp.stack([g0.reshape(-1, 32), g1.reshape(-1, 32)], axis=1)


@functools.partial(jax.jit, static_argnums=(3,))
def _run(x_flat, codes, cb_packed, total_words):
    words_per_tile = total_words // _NW
    chunk = min(_CHUNK, words_per_tile)
    n_chunks = words_per_tile // chunk
    mesh = plsc.VectorSubcoreMesh(core_axis_name="c", subcore_axis_name="s")

    @functools.partial(
        pl.kernel,
        out_type=jax.ShapeDtypeStruct((total_words, _D), jnp.float32),
        mesh=mesh,
        compiler_params=pltpu.CompilerParams(
            needs_layout_passes=False, use_tc_tiling_on_sc=False),
        scratch_types=[
            pltpu.VMEM((chunk,), jnp.int32),
            pltpu.VMEM((chunk * _M,), jnp.int32),
            pltpu.VMEM((chunk * _M,), jnp.int32),
            pltpu.VMEM((chunk * _M, 2, 32), jnp.bfloat16),
            pltpu.VMEM((chunk, _D), jnp.float32),
            pltpu.SemaphoreType.DMA,
            pltpu.SemaphoreType.DMA,
        ],
    )
    def run(x_hbm, codes_hbm, cbp_hbm, out_hbm, x_v, idx_v, codes_v, rows_v,
            out_v, sem1, sem2):
        wid = lax.axis_index("s") * _NC + lax.axis_index("c")
        base = wid * words_per_tile
        lane = lax.iota(jnp.int32, 16)
        sub_w = lax.shift_right_logical(lane, 3)
        sub_m = lax.bitwise_and(lane, 7)

        def chunk_body(ci, carry):
            gbase = base + ci * chunk
            pltpu.sync_copy(x_hbm.at[pl.ds(gbase, chunk)], x_v)

            def idx_body(j, c2):
                xg = plsc.load_gather(x_v, [2 * j + sub_w])
                idx_v[pl.ds(16 * j, 16)] = xg * _M + sub_m
                return c2

            lax.fori_loop(0, chunk * _M // 16, idx_body, 0)
            pltpu.async_copy(codes_hbm.at[idx_v], codes_v, sem1).wait()
            pltpu.async_copy(cbp_hbm.at[codes_v], rows_v, sem2).wait()

            def word_body(w, c2):
                r = w * _M
                for g in range(2):
                    s0 = rows_v[r + 0, g, :] + rows_v[r + 1, g, :]
                    s1 = rows_v[r + 2, g, :] + rows_v[r + 3, g, :]
                    s2 = rows_v[r + 4, g, :] + rows_v[r + 5, g, :]
                    s3 = rows_v[r + 6, g, :] + rows_v[r + 7, g, :]
                    acc = (s0 + s1) + (s2 + s3)
                    a, b = plsc.unpack(acc, format=_FMT)
                    out_v[w, pl.ds(32 * g, 16)] = a
                    out_v[w, pl.ds(32 * g + 16, 16)] = b
                return c2

            lax.fori_loop(0, chunk, word_body, 0)
            pltpu.sync_copy(out_v, out_hbm.at[pl.ds(gbase, chunk)])
            return carry

        lax.fori_loop(0, n_chunks, chunk_body, 0)

    return run(x_flat, codes, cb_packed)


def kernel(x, codes, codebook):
    bsz, seq = x.shape
    total = bsz * seq
    x_flat = x.reshape(total).astype(jnp.int32)
    codes_flat = codes.reshape(-1)
    cb_packed = _pack_codebook(codebook)
    out = _run(x_flat, codes_flat, cb_packed, total)
    return out.reshape(bsz, seq, _D)


# speedup vs baseline: 8.8153x; 1.2530x over previous
import functools

import jax
import jax.numpy as jnp
from jax import lax
from jax.experimental import pallas as pl
from jax.experimental.pallas import tpu as pltpu
from jax.experimental.pallas import tpu_sc as plsc

_NC, _NS = 2, 16
_NW = _NC * _NS
_D = 64
_M = 8
_CHUNK = 128


def _pack_codebook(codebook):
    cb16 = codebook.astype(jnp.bfloat16)
    g0 = jnp.stack([cb16[:, 0:16], cb16[:, 16:32]], axis=-1)
    g1 = jnp.stack([cb16[:, 32:48], cb16[:, 48:64]], axis=-1)
    return jnp.stack([g0.reshape(-1, 32), g1.reshape(-1, 32)], axis=1)


@functools.partial(jax.jit, static_argnums=(3,))
def _run(x_flat, codes, cb_packed, total_words):
    wpt = total_words // _NW
    chunk = min(_CHUNK, wpt)
    n_chunks = wpt // chunk
    assert n_chunks % 2 == 0 or n_chunks == 1
    mesh = plsc.VectorSubcoreMesh(core_axis_name="c", subcore_axis_name="s")

    @functools.partial(
        pl.kernel,
        out_type=jax.ShapeDtypeStruct((total_words, _D), jnp.float32),
        mesh=mesh,
        compiler_params=pltpu.CompilerParams(
            needs_layout_passes=False, use_tc_tiling_on_sc=False),
        scratch_types=[
            pltpu.VMEM((wpt,), jnp.int32),
            pltpu.VMEM((chunk, _M), jnp.int32),
            pltpu.VMEM((chunk, _M), jnp.int32),
            pltpu.VMEM((chunk * _M,), jnp.int32),
            pltpu.VMEM((chunk * _M,), jnp.int32),
            pltpu.VMEM((chunk * _M, 2, 32), jnp.bfloat16),
            pltpu.VMEM((chunk * _M, 2, 32), jnp.bfloat16),
            pltpu.VMEM((chunk, _D), jnp.float32),
            pltpu.VMEM((chunk, _D), jnp.float32),
            pltpu.SemaphoreType.DMA,
            pltpu.SemaphoreType.DMA,
            pltpu.SemaphoreType.DMA,
            pltpu.SemaphoreType.DMA,
            pltpu.SemaphoreType.DMA,
            pltpu.SemaphoreType.DMA,
        ],
    )
    def run(x_hbm, codes_hbm, cbp_hbm, out_hbm, x_all,
            codes0, codes1, flat0, flat1, rows0, rows1, outv0, outv1,
            semc0, semc1, semr0, semr1, semo0, semo1):
        codes_b = (codes0, codes1)
        flat_b = (flat0, flat1)
        rows_b = (rows0, rows1)
        out_b = (outv0, outv1)
        semc = (semc0, semc1)
        semr = (semr0, semr1)
        semo = (semo0, semo1)

        wid = lax.axis_index("s") * _NC + lax.axis_index("c")
        base = wid * wpt
        pltpu.sync_copy(x_hbm.at[pl.ds(base, wpt)], x_all)

        lane = lax.iota(jnp.int32, 16)
        sub_w = lax.shift_right_logical(lane, 3)
        sub_m = lax.bitwise_and(lane, 7)

        def codes_gather(k, b):
            return pltpu.make_async_copy(
                codes_hbm.at[x_all.at[pl.ds(k * chunk, chunk)]],
                codes_b[b], semc[b])

        def rows_gather(b):
            return pltpu.make_async_copy(
                cbp_hbm.at[flat_b[b]], rows_b[b], semr[b])

        def out_copy(k, b):
            return pltpu.make_async_copy(
                out_b[b], out_hbm.at[pl.ds(base + k * chunk, chunk)], semo[b])

        def s1(i, b):
            codes_gather(i, b).start()

        def s2(k, b):
            codes_gather(k, b).wait()

            def flat_body(j, c2):
                v = plsc.load_gather(codes_b[b], [2 * j + sub_w, sub_m])
                flat_b[b][pl.ds(16 * j, 16)] = v
                return c2

            lax.fori_loop(0, chunk * _M // 16, flat_body, 0)
            rows_gather(b).start()

        def s3(k, b):
            rows_gather(b).wait()

            @pl.when(k >= 2)
            def _():
                out_copy(k - 2, b).wait()

            rows = rows_b[b]
            outv = out_b[b]

            def word_body(w, c2):
                r = w * _M
                for g in range(2):
                    s0 = rows[r + 0, g, :] + rows[r + 1, g, :]
                    t0 = rows[r + 2, g, :] + rows[r + 3, g, :]
                    s1_ = rows[r + 4, g, :] + rows[r + 5, g, :]
                    t1 = rows[r + 6, g, :] + rows[r + 7, g, :]
                    acc = (s0 + t0) + (s1_ + t1)
                    a, b_ = plsc.unpack(acc, format=plsc.PackFormat.INTERLEAVED)
                    outv[w, pl.ds(32 * g, 16)] = a
                    outv[w, pl.ds(32 * g + 16, 16)] = b_
                return c2

            lax.fori_loop(0, chunk, word_body, 0)
            out_copy(k, b).start()

        def pair_body(ii, carry):
            for u in (0, 1):
                i = 2 * ii + u

                @pl.when(i < n_chunks)
                def _():
                    s1(i, u)

                @pl.when(jnp.logical_and(i >= 1, i <= n_chunks))
                def _():
                    s2(i - 1, 1 - u)

                @pl.when(jnp.logical_and(i >= 2, i <= n_chunks + 1))
                def _():
                    s3(i - 2, u)
            return carry

        lax.fori_loop(0, (n_chunks + 2) // 2, pair_body, 0)
        out_copy(n_chunks - 2, 0).wait()
        out_copy(n_chunks - 1, 1).wait()

    return run(x_flat, codes, cb_packed)


def kernel(x, codes, codebook):
    bsz, seq = x.shape
    total = bsz * seq
    x_flat = x.reshape(total).astype(jnp.int32)
    cb_packed = _pack_codebook(codebook)
    out = _run(x_flat, codes, cb_packed, total)
    return out.reshape(bsz, seq, _D)
